# baseline (speedup 1.0000x reference)
import functools

import jax
import jax.numpy as jnp
from jax import lax
from jax.experimental import pallas as pl
from jax.experimental.pallas import tpu as pltpu
from jax.experimental.pallas import tpu_sc as plsc

_B, _C, _H, _W = 8, 19, 512, 512
_NC, _NS = 2, 16
_NW = _NC * _NS
_RPW = (_B * _H) // _NW
_SR = 8
_NSLAB = _RPW // _SR
_QW = 128
_NQ = _W // _QW
_NSTEP = _NSLAB * _NQ
_NBUF = 5

_mesh = plsc.VectorSubcoreMesh(core_axis_name="c", subcore_axis_name="s")


@functools.partial(
    pl.kernel,
    out_type=jax.ShapeDtypeStruct((_NW, 16), jnp.float32),
    mesh=_mesh,
    compiler_params=pltpu.CompilerParams(needs_layout_passes=False),
    scratch_types=[
        pltpu.VMEM((_NBUF, _C, _SR, _QW), jnp.float32),
        pltpu.VMEM((_SR, _W), jnp.int32),
        pltpu.VMEM((_SR, _W), jnp.int32),
        pltpu.VMEM((16,), jnp.float32),
        pltpu.SemaphoreType.DMA((_NBUF,)),
        pltpu.SemaphoreType.DMA,
        pltpu.SemaphoreType.DMA,
    ],
)
def _partials(z_hbm, l_hbm, c_hbm, out_hbm,
              zq, lb, cb, acc_v, sem_z, sem_l, sem_c):
    cid = lax.axis_index("c")
    sid = lax.axis_index("s")
    wid = sid * _NC + cid
    b = wid // 4
    h_base = (wid % 4) * _RPW
    iota = lax.iota(jnp.int32, 16)

    def issue_z(q, slot):
        s, w0 = q // _NQ, (q % _NQ) * _QW
        h0 = h_base + s * _SR
        return pltpu.async_copy(
            z_hbm.at[b, :, pl.ds(h0, _SR), pl.ds(w0, _QW)],
            zq.at[slot], sem_z.at[slot])

    def issue_lc(s):
        h0 = h_base + s * _SR
        dl = pltpu.async_copy(l_hbm.at[b, pl.ds(h0, _SR), :], lb, sem_l)
        dc = pltpu.async_copy(c_hbm.at[b, pl.ds(h0, _SR), :], cb, sem_c)
        return dl, dc

    lc_pend = {0: issue_lc(0)}
    z_pend = {}
    for q in range(_NBUF - 1):
        z_pend[q] = issue_z(q, q)

    total = jnp.zeros((16,), jnp.float32)
    for q in range(_NSTEP):
        slot = q % _NBUF
        s, w0 = q // _NQ, (q % _NQ) * _QW
        if q + _NBUF - 1 < _NSTEP:
            z_pend[q + _NBUF - 1] = issue_z(q + _NBUF - 1,
                                            (q + _NBUF - 1) % _NBUF)
        if q % _NQ == 0:
            dl, dc = lc_pend.pop(s)
            dl.wait()
            dc.wait()
        z_pend.pop(q).wait()
        zref = zq.at[slot]

        def body(i, acc):
            hl = i >> 3
            jj = i & 7
            lv = lb[hl, pl.ds(w0 + jj * 16, 16)]
            cv = cb[hl, pl.ds(w0 + jj * 16, 16)]
            hvec = jnp.full((16,), hl, jnp.int32)
            wvec = jj * 16 + iota
            gv = plsc.load_gather(zref, [lv, hvec, wvec])
            return acc + jnp.where(cv > 0, gv, jnp.zeros((16,), jnp.float32))

        qacc = lax.fori_loop(0, (_SR * _QW) // 16, body,
                             jnp.zeros((16,), jnp.float32), unroll=4)
        total = total + qacc
        if q % _NQ == _NQ - 1 and s + 1 < _NSLAB:
            lc_pend[s + 1] = issue_lc(s + 1)

    acc_v[...] = total
    pltpu.sync_copy(acc_v, out_hbm.at[wid])


@functools.partial(
    pl.kernel,
    out_type=jax.ShapeDtypeStruct((16,), jnp.float32),
    mesh=_mesh,
    compiler_params=pltpu.CompilerParams(needs_layout_passes=False),
    scratch_types=[
        pltpu.VMEM((_NW, 16), jnp.float32),
        pltpu.VMEM((16,), jnp.float32),
        pltpu.SemaphoreType.DMA,
    ],
)
def _finish(part_hbm, out_hbm, p_v, o_v, sem):
    cid = lax.axis_index("c")
    sid = lax.axis_index("s")
    wid = sid * _NC + cid

    @pl.when(wid == 0)
    def _():
        pltpu.async_copy(part_hbm, p_v, sem).wait()
        acc = jnp.zeros((16,), jnp.float32)
        for i in range(_NW):
            acc = acc + p_v[i, :]
        s = plsc.cumsum(acc)[15]
        o_v[...] = jnp.full((16,), s, jnp.float32)
        pltpu.sync_copy(o_v, out_hbm)


def kernel(z, condition, l):
    cf = condition.astype(jnp.int32)
    parts = _partials(z, l, cf)
    out16 = _finish(parts)
    return out16[0]


# speedup vs baseline: 1.0934x; 1.0934x over previous
import functools

import jax
import jax.numpy as jnp
from jax import lax
from jax.experimental import pallas as pl
from jax.experimental.pallas import tpu as pltpu
from jax.experimental.pallas import tpu_sc as plsc

_B, _C, _H, _W = 8, 19, 512, 512
_NC, _NS = 2, 16
_NW = _NC * _NS
_RPW = (_B * _H) // _NW
_SR = 8
_NSLAB = _RPW // _SR
_QW = 128
_NQ = _W // _QW
_NBUF = _NQ

_mesh = plsc.VectorSubcoreMesh(core_axis_name="c", subcore_axis_name="s")


@functools.partial(
    pl.kernel,
    out_type=jax.ShapeDtypeStruct((_NW, 16), jnp.float32),
    mesh=_mesh,
    compiler_params=pltpu.CompilerParams(needs_layout_passes=False),
    scratch_types=[
        pltpu.VMEM((_NBUF, _C, _SR, _QW), jnp.float32),
        pltpu.VMEM((2, _SR, _W), jnp.int32),
        pltpu.VMEM((2, _SR, _W), jnp.int32),
        pltpu.VMEM((16,), jnp.float32),
        pltpu.SemaphoreType.DMA((_NBUF,)),
        pltpu.SemaphoreType.DMA((2,)),
        pltpu.SemaphoreType.DMA((2,)),
    ],
)
def _partials(z_hbm, l_hbm, c_hbm, out_hbm,
              zq, lb, cb, acc_v, sem_z, sem_l, sem_c):
    cid = lax.axis_index("c")
    sid = lax.axis_index("s")
    wid = sid * _NC + cid
    b = wid // 4
    h_base = (wid % 4) * _RPW
    iota = lax.iota(jnp.int32, 16)

    def z_desc(s, k):
        h0 = h_base + s * _SR
        return pltpu.make_async_copy(
            z_hbm.at[b, :, pl.ds(h0, _SR), pl.ds(k * _QW, _QW)],
            zq.at[k], sem_z.at[k])

    def l_desc(s):
        h0 = h_base + s * _SR
        return pltpu.make_async_copy(
            l_hbm.at[b, pl.ds(h0, _SR), :], lb.at[s % 2], sem_l.at[s % 2])

    def c_desc(s):
        h0 = h_base + s * _SR
        return pltpu.make_async_copy(
            c_hbm.at[b, pl.ds(h0, _SR), :], cb.at[s % 2], sem_c.at[s % 2])

    l_desc(0).start()
    c_desc(0).start()
    for k in range(_NBUF):
        z_desc(0, k).start()

    def slab_body(s, total):
        @pl.when(s + 1 < _NSLAB)
        def _():
            l_desc(s + 1).start()
            c_desc(s + 1).start()
        l_desc(s).wait()
        c_desc(s).wait()

        for k in range(_NQ):
            z_desc(s, k).wait()
            zref = zq.at[k]

            def body(i, acc):
                hl = i >> 3
                jj = i & 7
                lv = lb[s % 2, hl, pl.ds(k * _QW + jj * 16, 16)]
                cv = cb[s % 2, hl, pl.ds(k * _QW + jj * 16, 16)]
                hvec = jnp.full((16,), hl, jnp.int32)
                wvec = jj * 16 + iota
                gv = plsc.load_gather(zref, [lv, hvec, wvec])
                return acc + jnp.where(cv > 0, gv,
                                       jnp.zeros((16,), jnp.float32))

            qacc = lax.fori_loop(0, (_SR * _QW) // 16, body,
                                 jnp.zeros((16,), jnp.float32))
            total = total + qacc

            @pl.when(s + 1 < _NSLAB)
            def _():
                z_desc(s + 1, k).start()
        return total

    total = lax.fori_loop(0, _NSLAB, slab_body,
                          jnp.zeros((16,), jnp.float32))
    acc_v[...] = total
    pltpu.sync_copy(acc_v, out_hbm.at[wid])


@functools.partial(
    pl.kernel,
    out_type=jax.ShapeDtypeStruct((16,), jnp.float32),
    mesh=_mesh,
    compiler_params=pltpu.CompilerParams(needs_layout_passes=False),
    scratch_types=[
        pltpu.VMEM((_NW, 16), jnp.float32),
        pltpu.VMEM((16,), jnp.float32),
        pltpu.SemaphoreType.DMA,
    ],
)
def _finish(part_hbm, out_hbm, p_v, o_v, sem):
    cid = lax.axis_index("c")
    sid = lax.axis_index("s")
    wid = sid * _NC + cid

    @pl.when(wid == 0)
    def _():
        pltpu.async_copy(part_hbm, p_v, sem).wait()
        acc = jnp.zeros((16,), jnp.float32)
        for i in range(_NW):
            acc = acc + p_v[i, :]
        s = plsc.cumsum(acc)[15]
        o_v[...] = jnp.full((16,), s, jnp.float32)
        pltpu.sync_copy(o_v, out_hbm)


def kernel(z, condition, l):
    cf = condition.astype(jnp.int32)
    parts = _partials(z, l, cf)
    out16 = _finish(parts)
    return out16[0]
